# baseline (speedup 1.0000x reference)
import jax
import jax.numpy as jnp
from jax.experimental import pallas as pl
from jax.experimental.pallas import tpu as pltpu


def _copy_kernel(in_ref, out_ref, vbuf, sem_in, sem_out):
    cin = pltpu.make_async_copy(in_ref, vbuf, sem_in)
    cin.start()
    cin.wait()
    cout = pltpu.make_async_copy(vbuf, out_ref, sem_out)
    cout.start()
    cout.wait()


def kernel(xyz, reconstruction, neighborsMatrix, numNeighbors, weightMatrix, arapWeight):
    n, d = reconstruction.shape
    flat = reconstruction.reshape(-1)
    out = pl.pallas_call(
        _copy_kernel,
        out_shape=jax.ShapeDtypeStruct(flat.shape, flat.dtype),
        in_specs=[pl.BlockSpec(memory_space=pltpu.MemorySpace.HBM)],
        out_specs=pl.BlockSpec(memory_space=pltpu.MemorySpace.HBM),
        scratch_shapes=[
            pltpu.VMEM(flat.shape, flat.dtype),
            pltpu.SemaphoreType.DMA,
            pltpu.SemaphoreType.DMA,
        ],
    )(flat)
    return out.reshape(n, d)
